# baseline (speedup 1.0000x reference)
import functools

import jax
import jax.numpy as jnp
from jax.experimental import pallas as pl
from jax.experimental.pallas import tpu as pltpu

_MAXI = 8192


def _lase_kernel(n, d, inv_np1, keep,
                 x_ref, w0_ref, w1_ref, wq_ref, wk_ref, wv_ref, wo_ref,
                 codes1_ref, n1_ref, codes2_ref, n2_ref, scale2_ref,
                 out_ref, xw1_ref, u_ref, vo_ref):
    x = x_ref[:]
    out_ref[:] = (jnp.dot(x, w0_ref[:], preferred_element_type=jnp.float32)
                  * inv_np1 + x * keep)
    xw1_ref[:] = (jnp.dot(x, w1_ref[:], preferred_element_type=jnp.float32)
                  * inv_np1)
    m = jnp.dot(wk_ref[:], wq_ref[:].T, preferred_element_type=jnp.float32)
    u_ref[:] = jnp.dot(x, m, preferred_element_type=jnp.float32)
    wvo = jnp.dot(wv_ref[:], wo_ref[:], preferred_element_type=jnp.float32)
    vo_ref[:] = jnp.dot(x, wvo, preferred_element_type=jnp.float32)

    def body1(e, carry):
        code = codes1_ref[e]
        s = code // n
        t = code - s * n
        out_ref[pl.ds(t, 1), :] += xw1_ref[pl.ds(s, 1), :]
        return carry

    jax.lax.fori_loop(0, n1_ref[0, 0], body1, 0)

    sc = scale2_ref[0, 0]

    def body2(e, carry):
        code = codes2_ref[e]
        s = code // n
        t = code - s * n
        w = jnp.sum(u_ref[pl.ds(s, 1), :] * x_ref[pl.ds(t, 1), :])
        out_ref[pl.ds(t, 1), :] -= (sc * w) * vo_ref[pl.ds(s, 1), :]
        return carry

    jax.lax.fori_loop(0, n2_ref[0, 0], body2, 0)


def kernel(input, edge_index, edge_index_2, mask, W0, W1, Wq, Wk, Wv, Wo):
    n, d = input.shape
    e = edge_index.shape[1]

    cm = (mask[0] * n + mask[1]).astype(jnp.int32)
    cms = jnp.sort(cm)

    def prep(ei):
        cs = jnp.sort((ei[0] * n + ei[1]).astype(jnp.int32))
        pos = jnp.searchsorted(cms, cs)
        member = (pos < e) & (cms[jnp.minimum(pos, e - 1)] == cs)
        uniq = jnp.concatenate(
            [jnp.ones((1,), bool), cs[1:] != cs[:-1]])
        f = member & uniq
        idx = jnp.nonzero(f, size=_MAXI, fill_value=e)[0]
        codes = jnp.where(idx < e, cs[jnp.clip(idx, 0, e - 1)], -1)
        cnt = jnp.sum(f).astype(jnp.int32)
        return codes.astype(jnp.int32), cnt.reshape(1, 1)

    codes1, n1 = prep(edge_index)
    codes2, n2 = prep(edge_index_2)

    np1 = float(e) / float(n)
    inv_np1 = 1.0 / np1
    keep = (np1 - 1.0) / np1
    scale2 = (n / jnp.sqrt(jnp.float32(d))
              / jnp.maximum(n2, 1).astype(jnp.float32)).reshape(1, 1)

    body = functools.partial(_lase_kernel, n, d, inv_np1, keep)
    smem = pl.BlockSpec(memory_space=pltpu.SMEM)
    return pl.pallas_call(
        body,
        out_shape=jax.ShapeDtypeStruct((n, d), jnp.float32),
        in_specs=[pl.BlockSpec(memory_space=pltpu.VMEM)] * 7
        + [smem, smem, smem, smem, smem],
        out_specs=pl.BlockSpec(memory_space=pltpu.VMEM),
        scratch_shapes=[
            pltpu.VMEM((n, d), jnp.float32),
            pltpu.VMEM((n, d), jnp.float32),
            pltpu.VMEM((n, d), jnp.float32),
        ],
    )(input.astype(jnp.float32), W0, W1, Wq, Wk, Wv, Wo,
      codes1, n1, codes2, n2, scale2)


# speedup vs baseline: 3.9984x; 3.9984x over previous
import functools

import jax
import jax.numpy as jnp
from jax.experimental import pallas as pl
from jax.experimental.pallas import tpu as pltpu

_MAXI = 8192


def _lase_kernel(n, d, inv_np1, keep,
                 x_ref, w0_ref, w1_ref, wq_ref, wk_ref, wv_ref, wo_ref,
                 codes1_ref, n1_ref, codes2_ref, n2_ref, scale2_ref,
                 out_ref, xw1_ref, u_ref, vo_ref):
    x = x_ref[:]
    out_ref[:] = (jnp.dot(x, w0_ref[:], preferred_element_type=jnp.float32)
                  * inv_np1 + x * keep)
    xw1_ref[:] = (jnp.dot(x, w1_ref[:], preferred_element_type=jnp.float32)
                  * inv_np1)
    m = jnp.dot(wk_ref[:], wq_ref[:].T, preferred_element_type=jnp.float32)
    u_ref[:] = jnp.dot(x, m, preferred_element_type=jnp.float32)
    wvo = jnp.dot(wv_ref[:], wo_ref[:], preferred_element_type=jnp.float32)
    vo_ref[:] = jnp.dot(x, wvo, preferred_element_type=jnp.float32)

    def body1(e, carry):
        code = codes1_ref[e]
        s = code // n
        t = code - s * n
        out_ref[pl.ds(t, 1), :] += xw1_ref[pl.ds(s, 1), :]
        return carry

    jax.lax.fori_loop(0, n1_ref[0, 0], body1, 0)

    sc = scale2_ref[0, 0]

    def body2(e, carry):
        code = codes2_ref[e]
        s = code // n
        t = code - s * n
        w = jnp.sum(u_ref[pl.ds(s, 1), :] * x_ref[pl.ds(t, 1), :])
        out_ref[pl.ds(t, 1), :] -= (sc * w) * vo_ref[pl.ds(s, 1), :]
        return carry

    jax.lax.fori_loop(0, n2_ref[0, 0], body2, 0)


def kernel(input, edge_index, edge_index_2, mask, W0, W1, Wq, Wk, Wv, Wo):
    n, d = input.shape
    e = edge_index.shape[1]

    big = jnp.int32(2147483647)
    cm = (mask[0] * n + mask[1]).astype(jnp.int32)
    bm = jnp.zeros((n * n,), jnp.int8).at[cm].set(1, mode='drop')

    def prep(ei):
        c = (ei[0] * n + ei[1]).astype(jnp.int32)
        f = bm[c] == 1
        idx = jnp.nonzero(f, size=_MAXI, fill_value=e)[0]
        codes = jnp.where(idx < e, c[jnp.clip(idx, 0, e - 1)], big)
        cs = jnp.sort(codes)
        uniq = jnp.concatenate(
            [jnp.ones((1,), bool), cs[1:] != cs[:-1]]) & (cs < big)
        idx2 = jnp.nonzero(uniq, size=_MAXI, fill_value=_MAXI - 1)[0]
        final = jnp.where(jnp.arange(_MAXI) < jnp.sum(uniq),
                          cs[jnp.clip(idx2, 0, _MAXI - 1)], -1)
        cnt = jnp.sum(uniq).astype(jnp.int32)
        return final.astype(jnp.int32), cnt.reshape(1, 1)

    codes1, n1 = prep(edge_index)
    codes2, n2 = prep(edge_index_2)

    np1 = float(e) / float(n)
    inv_np1 = 1.0 / np1
    keep = (np1 - 1.0) / np1
    scale2 = (n / jnp.sqrt(jnp.float32(d))
              / jnp.maximum(n2, 1).astype(jnp.float32)).reshape(1, 1)

    body = functools.partial(_lase_kernel, n, d, inv_np1, keep)
    smem = pl.BlockSpec(memory_space=pltpu.SMEM)
    return pl.pallas_call(
        body,
        out_shape=jax.ShapeDtypeStruct((n, d), jnp.float32),
        in_specs=[pl.BlockSpec(memory_space=pltpu.VMEM)] * 7
        + [smem, smem, smem, smem, smem],
        out_specs=pl.BlockSpec(memory_space=pltpu.VMEM),
        scratch_shapes=[
            pltpu.VMEM((n, d), jnp.float32),
            pltpu.VMEM((n, d), jnp.float32),
            pltpu.VMEM((n, d), jnp.float32),
        ],
    )(input.astype(jnp.float32), W0, W1, Wq, Wk, Wv, Wo,
      codes1, n1, codes2, n2, scale2)


# speedup vs baseline: 4.0833x; 1.0212x over previous
import functools

import jax
import jax.numpy as jnp
from jax.experimental import pallas as pl
from jax.experimental.pallas import tpu as pltpu

_MAXI = 8192


def _lase_kernel(n, d, inv_np1, keep,
                 x_ref, w0_ref, w1_ref, wq_ref, wk_ref, wv_ref, wo_ref,
                 codes1_ref, n1_ref, codes2_ref, n2_ref, scale2_ref,
                 out_ref, xw1_ref, u_ref, vo_ref):
    x = x_ref[:]
    out_ref[:] = (jnp.dot(x, w0_ref[:], preferred_element_type=jnp.float32)
                  * inv_np1 + x * keep)
    xw1_ref[:] = (jnp.dot(x, w1_ref[:], preferred_element_type=jnp.float32)
                  * inv_np1)
    m = jnp.dot(wk_ref[:], wq_ref[:].T, preferred_element_type=jnp.float32)
    u_ref[:] = jnp.dot(x, m, preferred_element_type=jnp.float32)
    wvo = jnp.dot(wv_ref[:], wo_ref[:], preferred_element_type=jnp.float32)
    vo_ref[:] = jnp.dot(x, wvo, preferred_element_type=jnp.float32)

    def body1(e, carry):
        code = codes1_ref[e]
        s = code // n
        t = code - s * n
        out_ref[pl.ds(t, 1), :] += xw1_ref[pl.ds(s, 1), :]
        return carry

    jax.lax.fori_loop(0, n1_ref[0, 0], body1, 0)

    sc = scale2_ref[0, 0]

    def body2(e, carry):
        code = codes2_ref[e]
        s = code // n
        t = code - s * n
        w = jnp.sum(u_ref[pl.ds(s, 1), :] * x_ref[pl.ds(t, 1), :])
        out_ref[pl.ds(t, 1), :] -= (sc * w) * vo_ref[pl.ds(s, 1), :]
        return carry

    jax.lax.fori_loop(0, n2_ref[0, 0], body2, 0)


def kernel(input, edge_index, edge_index_2, mask, W0, W1, Wq, Wk, Wv, Wo):
    n, d = input.shape
    e = edge_index.shape[1]

    big = jnp.int32(2147483647)
    cm = (mask[0] * n + mask[1]).astype(jnp.int32)
    bm = jnp.zeros((n * n,), jnp.int8).at[cm].set(1, mode='drop')

    c1 = (edge_index[0] * n + edge_index[1]).astype(jnp.int32)
    c2 = (edge_index_2[0] * n + edge_index_2[1]).astype(jnp.int32)
    fb = bm[jnp.concatenate([c1, c2])] == 1

    def prep(c, f):
        idx = jnp.nonzero(f, size=_MAXI, fill_value=e)[0]
        codes = jnp.where(idx < e, c[jnp.clip(idx, 0, e - 1)], big)
        cs = jnp.sort(codes)
        uniq = jnp.concatenate(
            [jnp.ones((1,), bool), cs[1:] != cs[:-1]]) & (cs < big)
        idx2 = jnp.nonzero(uniq, size=_MAXI, fill_value=_MAXI - 1)[0]
        final = jnp.where(jnp.arange(_MAXI) < jnp.sum(uniq),
                          cs[jnp.clip(idx2, 0, _MAXI - 1)], -1)
        cnt = jnp.sum(uniq).astype(jnp.int32)
        return final.astype(jnp.int32), cnt.reshape(1, 1)

    codes1, n1 = prep(c1, fb[:e])
    codes2, n2 = prep(c2, fb[e:])

    np1 = float(e) / float(n)
    inv_np1 = 1.0 / np1
    keep = (np1 - 1.0) / np1
    scale2 = (n / jnp.sqrt(jnp.float32(d))
              / jnp.maximum(n2, 1).astype(jnp.float32)).reshape(1, 1)

    body = functools.partial(_lase_kernel, n, d, inv_np1, keep)
    smem = pl.BlockSpec(memory_space=pltpu.SMEM)
    return pl.pallas_call(
        body,
        out_shape=jax.ShapeDtypeStruct((n, d), jnp.float32),
        in_specs=[pl.BlockSpec(memory_space=pltpu.VMEM)] * 7
        + [smem, smem, smem, smem, smem],
        out_specs=pl.BlockSpec(memory_space=pltpu.VMEM),
        scratch_shapes=[
            pltpu.VMEM((n, d), jnp.float32),
            pltpu.VMEM((n, d), jnp.float32),
            pltpu.VMEM((n, d), jnp.float32),
        ],
    )(input.astype(jnp.float32), W0, W1, Wq, Wk, Wv, Wo,
      codes1, n1, codes2, n2, scale2)
